# baseline (speedup 1.0000x reference)
import jax
import jax.numpy as jnp
from jax import lax
from jax.experimental import pallas as pl
from jax.experimental.pallas import tpu as pltpu
from jax.experimental.pallas import tpu_sc as plsc

N = 10000
E = 640000
EMB = 256
D = 128
NC_OUT = 41476

NCORE = 2
NSUB = 16
NW = NCORE * NSUB
EPW = E // NW
CH = 128
FULL_CHUNKS = EPW // CH
TAIL = EPW - FULL_CHUNKS * CH
RPS = 624
RTAIL = N - NSUB * RPS
NP = 10240
RPS1 = NP // NSUB

_mesh = plsc.VectorSubcoreMesh(core_axis_name="c", subcore_axis_name="s")


def _wid():
    return lax.axis_index("s") * NCORE + lax.axis_index("c")


def _degree_body(row_h, col_h, ones_h, zer_h, outc_h, outr_h,
                 rv, cv, rvt, cvt, ones_v, bnc, acc_c, acc_r, sem):
    cid = lax.axis_index("c")
    sid = lax.axis_index("s")
    w = _wid()
    pltpu.sync_copy(ones_h, ones_v)
    pltpu.sync_copy(zer_h, bnc)
    pltpu.sync_copy(bnc, acc_c.at[pl.ds(sid * RPS1, RPS1)])
    pltpu.sync_copy(bnc, acc_r.at[pl.ds(sid * RPS1, RPS1)])
    plsc.subcore_barrier()
    base_w = w * EPW

    def chunk(k, _):
        b = base_w + k * CH
        pltpu.sync_copy(row_h.at[pl.ds(b, CH)], rv)
        pltpu.sync_copy(col_h.at[pl.ds(b, CH)], cv)
        pltpu.sync_copy(ones_v.at[pl.ds(0, CH)], acc_c.at[cv], add=True)
        pltpu.sync_copy(ones_v.at[pl.ds(0, CH)], acc_r.at[rv], add=True)
        return _

    lax.fori_loop(0, FULL_CHUNKS, chunk, None)
    b = base_w + FULL_CHUNKS * CH
    pltpu.sync_copy(row_h.at[pl.ds(b, TAIL)], rvt)
    pltpu.sync_copy(col_h.at[pl.ds(b, TAIL)], cvt)
    pltpu.sync_copy(ones_v.at[pl.ds(0, TAIL)], acc_c.at[cvt], add=True)
    pltpu.sync_copy(ones_v.at[pl.ds(0, TAIL)], acc_r.at[rvt], add=True)
    plsc.subcore_barrier()
    r0 = sid * RPS1
    pltpu.sync_copy(acc_c.at[pl.ds(r0, RPS1)], bnc)
    pltpu.sync_copy(bnc, outc_h.at[cid, 0, pl.ds(r0, RPS1)])
    pltpu.sync_copy(acc_r.at[pl.ds(r0, RPS1)], bnc)
    pltpu.sync_copy(bnc, outr_h.at[cid, 0, pl.ds(r0, RPS1)])


def _sc_degrees(row, col):
    ones = jnp.ones((CH,), jnp.float32)
    zer = jnp.zeros((RPS1,), jnp.float32)
    k = pl.kernel(
        _degree_body,
        out_type=(jax.ShapeDtypeStruct((NCORE, 1, NP), jnp.float32),
                  jax.ShapeDtypeStruct((NCORE, 1, NP), jnp.float32)),
        mesh=_mesh,
        scratch_types=[
            pltpu.VMEM((CH,), jnp.int32), pltpu.VMEM((CH,), jnp.int32),
            pltpu.VMEM((TAIL,), jnp.int32), pltpu.VMEM((TAIL,), jnp.int32),
            pltpu.VMEM((CH,), jnp.float32), pltpu.VMEM((RPS1,), jnp.float32),
            pltpu.VMEM_SHARED((NP,), jnp.float32),
            pltpu.VMEM_SHARED((NP,), jnp.float32),
            pltpu.SemaphoreType.DMA,
        ],
    )
    return k(row, col, ones, zer)


FULL_ROW_CH = N // CH
ROW_TAIL = N - FULL_ROW_CH * CH


def _embed_body(iu_h, ii_h, uemb_h, iemb_h, outu_h, outi_h,
                iv, ubuf, ivt, ubuft, sem):
    w = _wid()

    def chunk(ch):
        b = ch * CH
        pltpu.sync_copy(iu_h.at[pl.ds(b, CH)], iv)
        pltpu.async_copy(uemb_h.at[iv], ubuf, sem).wait()
        pltpu.sync_copy(ubuf, outu_h.at[pl.ds(b, CH), :])
        pltpu.sync_copy(ii_h.at[pl.ds(b, CH)], iv)
        pltpu.async_copy(iemb_h.at[iv], ubuf, sem).wait()
        pltpu.sync_copy(ubuf, outi_h.at[pl.ds(b, CH), :])

    def it(k, _):
        ch = k * NW + w

        @pl.when(ch < FULL_ROW_CH)
        def _do():
            chunk(ch)
        return _

    lax.fori_loop(0, (FULL_ROW_CH + NW - 1) // NW, it, None)

    @pl.when(w == FULL_ROW_CH % NW)
    def _tail():
        b = FULL_ROW_CH * CH
        pltpu.sync_copy(iu_h.at[pl.ds(b, ROW_TAIL)], ivt)
        pltpu.async_copy(uemb_h.at[ivt], ubuft, sem).wait()
        pltpu.sync_copy(ubuft, outu_h.at[pl.ds(b, ROW_TAIL), :])
        pltpu.sync_copy(ii_h.at[pl.ds(b, ROW_TAIL)], ivt)
        pltpu.async_copy(iemb_h.at[ivt], ubuft, sem).wait()
        pltpu.sync_copy(ubuft, outi_h.at[pl.ds(b, ROW_TAIL), :])


def _sc_embed(iu, ii, uemb, iemb):
    k = pl.kernel(
        _embed_body,
        out_type=(jax.ShapeDtypeStruct((N, EMB), jnp.float32),
                  jax.ShapeDtypeStruct((N, EMB), jnp.float32)),
        mesh=_mesh,
        scratch_types=[
            pltpu.VMEM((CH,), jnp.int32), pltpu.VMEM((CH, EMB), jnp.float32),
            pltpu.VMEM((ROW_TAIL,), jnp.int32), pltpu.VMEM((ROW_TAIL, EMB), jnp.float32),
            pltpu.SemaphoreType.DMA,
        ],
    )
    return k(iu, ii, uemb, iemb)


CPW = 160
NJ = N + 16


def _seg_pipeline(x_h, acc, row3_h, col3_h, base, riv, civ, xb,
                  isems, gsems, ssems, per_chunk=None, extra_gather=None,
                  extra_wait=None, extra_scatter=None, extra_swait=None):
    def ifetch(k, m):
        pltpu.async_copy(row3_h.at[base + k, 0, :], riv[m], isems[m])
        pltpu.async_copy(col3_h.at[base + k, 0, :], civ[m], isems[m])

    def iwait(k, m):
        pltpu.make_async_copy(row3_h.at[base + k, 0, :], riv[m], isems[m]).wait()
        pltpu.make_async_copy(col3_h.at[base + k, 0, :], civ[m], isems[m]).wait()

    def gfire(k, p, m):
        pltpu.async_copy(x_h.at[riv[m]], xb[p], gsems[p])
        if extra_gather is not None:
            extra_gather(k, p, m)

    def gwait(k, p, m):
        pltpu.make_async_copy(x_h.at[riv[m]], xb[p], gsems[p]).wait()
        if extra_wait is not None:
            extra_wait(k, p, m)

    def sfire(k, p, m):
        if per_chunk is not None:
            per_chunk(k, p, m)
        pltpu.async_copy(xb[p], acc.at[civ[m]], ssems[p], add=True)
        if extra_scatter is not None:
            extra_scatter(k, p, m)

    def swait(k, p, m):
        pltpu.make_async_copy(xb[p], acc.at[civ[m]], ssems[p]).wait()
        if extra_swait is not None:
            extra_swait(k, p, m)

    ifetch(0, 0)
    ifetch(1, 1)

    def body(kk, _):
        for i in range(4):
            k = 4 * kk + i
            p = i % 2
            m = i

            @pl.when(k >= 2)
            def _free():
                swait(k - 2, p, m)
            iwait(k, m)
            gfire(k, p, m)

            @pl.when(k >= 1)
            def _prev():
                gwait(k - 1, 1 - p, (i - 1) % 4)
                sfire(k - 1, 1 - p, (i - 1) % 4)

            @pl.when(k + 2 < CPW)
            def _pref():
                ifetch(k + 2, (i + 2) % 4)
        return _

    lax.fori_loop(0, CPW // 4, body, None)
    gwait(CPW - 1, 1, 3)
    sfire(CPW - 1, 1, 3)
    swait(CPW - 2, 0, 2)
    swait(CPW - 1, 1, 3)


def _segsum_body(x_h, row3_h, col3_h, zer_h, out_h,
                 riv0, riv1, riv2, riv3, civ0, civ1, civ2, civ3,
                 xb0, xb1, acc,
                 is0, is1, is2, is3, gs0, gs1, ss0, ss1):
    cid = lax.axis_index("c")
    sid = lax.axis_index("s")
    w = _wid()
    pltpu.sync_copy(zer_h, acc.at[pl.ds(sid * RPS, RPS), :])

    @pl.when(sid == NSUB - 1)
    def _z_tail():
        pltpu.sync_copy(zer_h.at[pl.ds(0, RTAIL), :], acc.at[pl.ds(NSUB * RPS, RTAIL), :])
    plsc.subcore_barrier()
    _seg_pipeline(x_h, acc, row3_h, col3_h, w * CPW,
                  (riv0, riv1, riv2, riv3), (civ0, civ1, civ2, civ3),
                  (xb0, xb1), (is0, is1, is2, is3), (gs0, gs1), (ss0, ss1))
    plsc.subcore_barrier()
    r0 = sid * RPS
    pltpu.sync_copy(acc.at[pl.ds(r0, RPS), :], out_h.at[cid, pl.ds(r0, RPS), :])

    @pl.when(sid == NSUB - 1)
    def _o_tail():
        t0 = NSUB * RPS
        pltpu.sync_copy(acc.at[pl.ds(t0, RTAIL), :], out_h.at[cid, pl.ds(t0, RTAIL), :])


def _idx_scratch():
    return [pltpu.VMEM((CH,), jnp.int32) for _ in range(8)]


def _sc_segsum(x, row3, col3):
    zer = jnp.zeros((RPS, D), jnp.float32)
    k = pl.kernel(
        _segsum_body,
        out_type=jax.ShapeDtypeStruct((NCORE, N, D), jnp.float32),
        mesh=_mesh,
        scratch_types=_idx_scratch() + [
            pltpu.VMEM((CH, D), jnp.float32), pltpu.VMEM((CH, D), jnp.float32),
            pltpu.VMEM_SHARED((NJ, D), jnp.float32),
        ] + [pltpu.SemaphoreType.DMA] * 8,
    )
    return k(x, row3, col3, zer)


def _gat_body(xw_h, als_h, ald_h, sv_h, row3_h, col3_h, zer_h, zer1_h, out_h, outd_h,
              riv0, riv1, riv2, riv3, civ0, civ1, civ2, civ3,
              xb0, xb1, asb0, asb1, adb0, adb1, exb0, exb1, sv_v, bnc, acc, den,
              is0, is1, is2, is3, gs0, gs1, ss0, ss1, ds0, ds1):
    cid = lax.axis_index("c")
    sid = lax.axis_index("s")
    w = _wid()
    pltpu.sync_copy(sv_h, sv_v)
    pltpu.sync_copy(zer_h, acc.at[pl.ds(sid * RPS, RPS), :])
    pltpu.sync_copy(zer1_h, bnc)
    pltpu.sync_copy(bnc, den.at[pl.ds(sid * RPS1, RPS1)])

    @pl.when(sid == NSUB - 1)
    def _z_tail():
        pltpu.sync_copy(zer_h.at[pl.ds(0, RTAIL), :], acc.at[pl.ds(NSUB * RPS, RTAIL), :])
    plsc.subcore_barrier()
    svec = sv_v[...]
    riv = (riv0, riv1, riv2, riv3)
    civ = (civ0, civ1, civ2, civ3)
    xb = (xb0, xb1)
    asb = (asb0, asb1)
    adb = (adb0, adb1)
    exb = (exb0, exb1)
    gsems = (gs0, gs1)
    ssems = (ss0, ss1)
    dsems = (ds0, ds1)

    def extra_gather(k, p, m):
        pltpu.async_copy(als_h.at[riv[m]], asb[p], gsems[p])
        pltpu.async_copy(ald_h.at[civ[m]], adb[p], gsems[p])

    def extra_wait(k, p, m):
        pltpu.make_async_copy(als_h.at[riv[m]], asb[p], gsems[p]).wait()
        pltpu.make_async_copy(ald_h.at[civ[m]], adb[p], gsems[p]).wait()

    def per_chunk(k, p, m):
        def grp(g, _):
            b16 = 16 * g
            z = asb[p][pl.ds(b16, 16)] + adb[p][pl.ds(b16, 16)]
            lr = jnp.where(z > 0, z, 0.2 * z)
            exv = jnp.exp(lr - svec)
            exb[p][pl.ds(b16, 16)] = exv
            for i2 in range(16):
                wv = exv[i2]
                for j in range(D // 16):
                    xb[p][b16 + i2, pl.ds(16 * j, 16)] = (
                        xb[p][b16 + i2, pl.ds(16 * j, 16)] * wv)
            return _

        lax.fori_loop(0, CH // 16, grp, None)

    def extra_scatter(k, p, m):
        pltpu.async_copy(exb[p], den.at[civ[m]], dsems[p], add=True)

    def extra_swait(k, p, m):
        pltpu.make_async_copy(exb[p], den.at[civ[m]], dsems[p]).wait()

    _seg_pipeline(xw_h, acc, row3_h, col3_h, w * CPW, riv, civ, xb,
                  (is0, is1, is2, is3), gsems, ssems,
                  per_chunk=per_chunk, extra_gather=extra_gather,
                  extra_wait=extra_wait, extra_scatter=extra_scatter,
                  extra_swait=extra_swait)
    plsc.subcore_barrier()
    r0 = sid * RPS
    pltpu.sync_copy(acc.at[pl.ds(r0, RPS), :], out_h.at[cid, pl.ds(r0, RPS), :])
    r1 = sid * RPS1
    pltpu.sync_copy(den.at[pl.ds(r1, RPS1)], bnc)
    pltpu.sync_copy(bnc, outd_h.at[cid, 0, pl.ds(r1, RPS1)])

    @pl.when(sid == NSUB - 1)
    def _o_tail():
        t0 = NSUB * RPS
        pltpu.sync_copy(acc.at[pl.ds(t0, RTAIL), :], out_h.at[cid, pl.ds(t0, RTAIL), :])


def _sc_gat(xw, als, ald, sv, row3, col3):
    zer = jnp.zeros((RPS, D), jnp.float32)
    zer1 = jnp.zeros((RPS1,), jnp.float32)
    k = pl.kernel(
        _gat_body,
        out_type=(jax.ShapeDtypeStruct((NCORE, N, D), jnp.float32),
                  jax.ShapeDtypeStruct((NCORE, 1, NP), jnp.float32)),
        mesh=_mesh,
        scratch_types=_idx_scratch() + [
            pltpu.VMEM((CH, D), jnp.float32), pltpu.VMEM((CH, D), jnp.float32),
            pltpu.VMEM((CH,), jnp.float32), pltpu.VMEM((CH,), jnp.float32),
            pltpu.VMEM((CH,), jnp.float32), pltpu.VMEM((CH,), jnp.float32),
            pltpu.VMEM((CH,), jnp.float32), pltpu.VMEM((CH,), jnp.float32),
            pltpu.VMEM((16,), jnp.float32), pltpu.VMEM((RPS1,), jnp.float32),
            pltpu.VMEM_SHARED((NJ, D), jnp.float32),
            pltpu.VMEM_SHARED((NP,), jnp.float32),
        ] + [pltpu.SemaphoreType.DMA] * 10,
    )
    return k(xw, als, ald, sv, row3, col3, zer, zer1)


_TC_PARAMS = pltpu.CompilerParams(vmem_limit_bytes=100 * 1024 * 1024)


def _tc(body, out_shape):
    return pl.pallas_call(body, out_shape=out_shape, compiler_params=_TC_PARAMS)


def _cnt(ref):
    return ref[0, 0, :N] + ref[1, 0, :N]


def _gcn_pre_body(u_r, i_r, wg_r, cnt_r, xw_r, xs_r):
    xw = jnp.dot(u_r[...], wg_r[:EMB, :], preferred_element_type=jnp.float32)
    xw = xw + jnp.dot(i_r[...], wg_r[EMB:, :], preferred_element_type=jnp.float32)
    dis = lax.rsqrt(_cnt(cnt_r) + 1.0)
    xw_r[...] = xw
    xs_r[...] = xw * dis[:, None]


def _gcn_post_body(agg_r, xw_r, cnt_r, gb_r, bng_r, bnb_r, h_r):
    agg = agg_r[0] + agg_r[1]
    xw = xw_r[...]
    dis = lax.rsqrt(_cnt(cnt_r) + 1.0)
    pre = dis[:, None] * (agg + dis[:, None] * xw) + gb_r[...]
    mean = jnp.mean(pre, axis=0)
    var = jnp.mean((pre - mean) ** 2, axis=0)
    y = (pre - mean) * lax.rsqrt(var + 1e-5) * bng_r[...] + bnb_r[...]
    h_r[...] = jnp.maximum(y, 0.0)


def _sage_body(s_r, h_r, cntc_r, wl_r, bl_r, wr_r, cntr_r, h2_r, xs2_r):
    s = s_r[0] + s_r[1]
    h = h_r[...]
    mean = s / jnp.clip(_cnt(cntc_r), 1.0, None)[:, None]
    h2 = jnp.dot(mean, wl_r[...], preferred_element_type=jnp.float32) + bl_r[...]
    h2 = h2 + jnp.dot(h, wr_r[...], preferred_element_type=jnp.float32)
    h2 = jnp.maximum(h2, 0.0)
    cr = _cnt(cntr_r)
    disr = jnp.where(cr > 0, lax.rsqrt(jnp.maximum(cr, 1e-30)), 0.0)
    h2_r[...] = h2
    xs2_r[...] = h2 * disr[:, None]


def _cheb_gat_prep_body(t_r, h_r, cntr_r, w0_r, w1_r, cb_r, gw_r, gas_r, gad_r,
                        xw_r, als_r, ald_r, sv_r):
    cr = _cnt(cntr_r)
    disr = jnp.where(cr > 0, lax.rsqrt(jnp.maximum(cr, 1e-30)), 0.0)
    tx1 = -disr[:, None] * (t_r[0] + t_r[1])
    h = h_r[...]
    h3 = jnp.dot(h, w0_r[...], preferred_element_type=jnp.float32)
    h3 = h3 + jnp.dot(tx1, w1_r[...], preferred_element_type=jnp.float32) + cb_r[...]
    h3 = jnp.maximum(h3, 0.0)
    xw = jnp.dot(h3, gw_r[...], preferred_element_type=jnp.float32)
    als = jnp.dot(xw, gas_r[...][:, None], preferred_element_type=jnp.float32)[:, 0]
    ald = jnp.dot(xw, gad_r[...][:, None], preferred_element_type=jnp.float32)[:, 0]
    s = jnp.maximum(jnp.max(als) + jnp.max(ald), 0.0)
    xw_r[...] = xw
    als_r[...] = als
    ald_r[...] = ald
    sv_r[...] = jnp.broadcast_to(s, (16,))


def _gat_finish(acc_r, den_r, xw_r, als_r, ald_r, gb_r):
    feat = acc_r[0] + acc_r[1]
    den = den_r[0, 0, :N] + den_r[1, 0, :N]
    als = als_r[...]
    ald = ald_r[...]
    s = jnp.maximum(jnp.max(als) + jnp.max(ald), 0.0)
    zs = als + ald
    exs = jnp.exp(jnp.where(zs > 0, zs, 0.2 * zs) - s)
    xw = xw_r[...]
    out = (feat + exs[:, None] * xw) / (den + exs)[:, None] + gb_r[...]
    return jnp.where(out > 0, out, jnp.exp(jnp.minimum(out, 0.0)) - 1.0)


def _gat1_post_body(acc_r, den_r, xw_r, als_r, ald_r, gb_r, gw2_r, gas2_r, gad2_r,
                    xw2_r, als2_r, ald2_r, sv2_r):
    h4 = _gat_finish(acc_r, den_r, xw_r, als_r, ald_r, gb_r)
    xw2 = jnp.dot(h4, gw2_r[...], preferred_element_type=jnp.float32)
    als2 = jnp.dot(xw2, gas2_r[...][:, None], preferred_element_type=jnp.float32)[:, 0]
    ald2 = jnp.dot(xw2, gad2_r[...][:, None], preferred_element_type=jnp.float32)[:, 0]
    s2 = jnp.maximum(jnp.max(als2) + jnp.max(ald2), 0.0)
    xw2_r[...] = xw2
    als2_r[...] = als2
    ald2_r[...] = ald2
    sv2_r[...] = jnp.broadcast_to(s2, (16,))


def _gat2_post_body(acc_r, den_r, xw_r, als_r, ald_r, gb_r, h5_r):
    h5_r[...] = _gat_finish(acc_r, den_r, xw_r, als_r, ald_r, gb_r)


BC = 256


def _final_body(h_r, w_r, b_r, o_r):
    o_r[...] = jnp.dot(h_r[...], w_r[...], preferred_element_type=jnp.float32) + b_r[...]


def _final_matmul(h5, pred_W, pred_b):
    nb = (NC_OUT + BC - 1) // BC
    return pl.pallas_call(
        _final_body,
        grid=(nb,),
        in_specs=[
            pl.BlockSpec((N, D), lambda j: (0, 0)),
            pl.BlockSpec((D, BC), lambda j: (0, j)),
            pl.BlockSpec((BC,), lambda j: (j,)),
        ],
        out_specs=pl.BlockSpec((N, BC), lambda j: (0, j)),
        out_shape=jax.ShapeDtypeStruct((N, NC_OUT), jnp.float32),
        compiler_params=_TC_PARAMS,
    )(h5, pred_W, pred_b)


def kernel(x, edge_index, user_emb, item_emb, gcn_W, gcn_b, bn_g, bn_b,
           sage_Wl, sage_bl, sage_Wr, cheb_W0, cheb_W1, cheb_b,
           gat1_W, gat1_as, gat1_ad, gat1_b, gat2_W, gat2_as, gat2_ad, gat2_b,
           pred_W, pred_b):
    row = edge_index[0]
    col = edge_index[1]
    npad = NW * CPW * CH - E
    row3 = jnp.concatenate([row, jnp.zeros((npad,), row.dtype)]).reshape(-1, 1, CH)
    col3 = jnp.concatenate([col, jnp.full((npad,), N, col.dtype)]).reshape(-1, 1, CH)

    cntc, cntr = _sc_degrees(row, col)
    u, i = _sc_embed(x[:, 0], x[:, 1], user_emb, item_emb)

    xw, xs = _tc(_gcn_pre_body,
                 (jax.ShapeDtypeStruct((N, D), jnp.float32),
                  jax.ShapeDtypeStruct((N, D), jnp.float32)))(u, i, gcn_W, cntc)
    agg = _sc_segsum(xs, row3, col3)
    h1 = _tc(_gcn_post_body, jax.ShapeDtypeStruct((N, D), jnp.float32))(
        agg, xw, cntc, gcn_b, bn_g, bn_b)

    s = _sc_segsum(h1, row3, col3)
    h2, xs2 = _tc(_sage_body,
                  (jax.ShapeDtypeStruct((N, D), jnp.float32),
                   jax.ShapeDtypeStruct((N, D), jnp.float32)))(
        s, h1, cntc, sage_Wl, sage_bl, sage_Wr, cntr)

    t = _sc_segsum(xs2, row3, col3)
    xw1, als1, ald1, sv1 = _tc(
        _cheb_gat_prep_body,
        (jax.ShapeDtypeStruct((N, D), jnp.float32),
         jax.ShapeDtypeStruct((N,), jnp.float32),
         jax.ShapeDtypeStruct((N,), jnp.float32),
         jax.ShapeDtypeStruct((16,), jnp.float32)))(
        t, h2, cntr, cheb_W0, cheb_W1, cheb_b, gat1_W, gat1_as, gat1_ad)

    acc1, den1 = _sc_gat(xw1, als1, ald1, sv1, row3, col3)
    xw2, als2, ald2, sv2 = _tc(
        _gat1_post_body,
        (jax.ShapeDtypeStruct((N, D), jnp.float32),
         jax.ShapeDtypeStruct((N,), jnp.float32),
         jax.ShapeDtypeStruct((N,), jnp.float32),
         jax.ShapeDtypeStruct((16,), jnp.float32)))(
        acc1, den1, xw1, als1, ald1, gat1_b, gat2_W, gat2_as, gat2_ad)

    acc2, den2 = _sc_gat(xw2, als2, ald2, sv2, row3, col3)
    h5 = _tc(_gat2_post_body, jax.ShapeDtypeStruct((N, D), jnp.float32))(
        acc2, den2, xw2, als2, ald2, gat2_b)

    return _final_matmul(h5, pred_W, pred_b)


# speedup vs baseline: 1.4784x; 1.4784x over previous
import jax
import jax.numpy as jnp
from jax import lax
from jax.experimental import pallas as pl
from jax.experimental.pallas import tpu as pltpu
from jax.experimental.pallas import tpu_sc as plsc

N = 10000
E = 640000
EMB = 256
D = 128
NC_OUT = 41476

NCORE = 2
NSUB = 16
NW = NCORE * NSUB
EPW = E // NW
CH = 128
FULL_CHUNKS = EPW // CH
TAIL = EPW - FULL_CHUNKS * CH
RPS = 624
RTAIL = N - NSUB * RPS
NP = 10240
RPS1 = NP // NSUB

_mesh = plsc.VectorSubcoreMesh(core_axis_name="c", subcore_axis_name="s")


def _wid():
    return lax.axis_index("s") * NCORE + lax.axis_index("c")


def _degree_body(row_h, col_h, ones_h, zer_h, outc_h, outr_h,
                 rv, cv, rvt, cvt, ones_v, bnc, acc_c, acc_r, sem):
    cid = lax.axis_index("c")
    sid = lax.axis_index("s")
    w = _wid()
    pltpu.sync_copy(ones_h, ones_v)
    pltpu.sync_copy(zer_h, bnc)
    pltpu.sync_copy(bnc, acc_c.at[pl.ds(sid * RPS1, RPS1)])
    pltpu.sync_copy(bnc, acc_r.at[pl.ds(sid * RPS1, RPS1)])
    plsc.subcore_barrier()
    base_w = w * EPW

    def chunk(k, _):
        b = base_w + k * CH
        pltpu.sync_copy(row_h.at[pl.ds(b, CH)], rv)
        pltpu.sync_copy(col_h.at[pl.ds(b, CH)], cv)
        pltpu.sync_copy(ones_v.at[pl.ds(0, CH)], acc_c.at[cv], add=True)
        pltpu.sync_copy(ones_v.at[pl.ds(0, CH)], acc_r.at[rv], add=True)
        return _

    lax.fori_loop(0, FULL_CHUNKS, chunk, None)
    b = base_w + FULL_CHUNKS * CH
    pltpu.sync_copy(row_h.at[pl.ds(b, TAIL)], rvt)
    pltpu.sync_copy(col_h.at[pl.ds(b, TAIL)], cvt)
    pltpu.sync_copy(ones_v.at[pl.ds(0, TAIL)], acc_c.at[cvt], add=True)
    pltpu.sync_copy(ones_v.at[pl.ds(0, TAIL)], acc_r.at[rvt], add=True)
    plsc.subcore_barrier()
    r0 = sid * RPS1
    pltpu.sync_copy(acc_c.at[pl.ds(r0, RPS1)], bnc)
    pltpu.sync_copy(bnc, outc_h.at[cid, 0, pl.ds(r0, RPS1)])
    pltpu.sync_copy(acc_r.at[pl.ds(r0, RPS1)], bnc)
    pltpu.sync_copy(bnc, outr_h.at[cid, 0, pl.ds(r0, RPS1)])


def _sc_degrees(row, col):
    ones = jnp.ones((CH,), jnp.float32)
    zer = jnp.zeros((RPS1,), jnp.float32)
    k = pl.kernel(
        _degree_body,
        out_type=(jax.ShapeDtypeStruct((NCORE, 1, NP), jnp.float32),
                  jax.ShapeDtypeStruct((NCORE, 1, NP), jnp.float32)),
        mesh=_mesh,
        scratch_types=[
            pltpu.VMEM((CH,), jnp.int32), pltpu.VMEM((CH,), jnp.int32),
            pltpu.VMEM((TAIL,), jnp.int32), pltpu.VMEM((TAIL,), jnp.int32),
            pltpu.VMEM((CH,), jnp.float32), pltpu.VMEM((RPS1,), jnp.float32),
            pltpu.VMEM_SHARED((NP,), jnp.float32),
            pltpu.VMEM_SHARED((NP,), jnp.float32),
            pltpu.SemaphoreType.DMA,
        ],
    )
    return k(row, col, ones, zer)


FULL_ROW_CH = N // CH
ROW_TAIL = N - FULL_ROW_CH * CH


def _embed_body(iu_h, ii_h, uemb_h, iemb_h, outu_h, outi_h,
                iv, ubuf, ivt, ubuft, sem):
    w = _wid()

    def chunk(ch):
        b = ch * CH
        pltpu.sync_copy(iu_h.at[pl.ds(b, CH)], iv)
        pltpu.async_copy(uemb_h.at[iv], ubuf, sem).wait()
        pltpu.sync_copy(ubuf, outu_h.at[pl.ds(b, CH), :])
        pltpu.sync_copy(ii_h.at[pl.ds(b, CH)], iv)
        pltpu.async_copy(iemb_h.at[iv], ubuf, sem).wait()
        pltpu.sync_copy(ubuf, outi_h.at[pl.ds(b, CH), :])

    def it(k, _):
        ch = k * NW + w

        @pl.when(ch < FULL_ROW_CH)
        def _do():
            chunk(ch)
        return _

    lax.fori_loop(0, (FULL_ROW_CH + NW - 1) // NW, it, None)

    @pl.when(w == FULL_ROW_CH % NW)
    def _tail():
        b = FULL_ROW_CH * CH
        pltpu.sync_copy(iu_h.at[pl.ds(b, ROW_TAIL)], ivt)
        pltpu.async_copy(uemb_h.at[ivt], ubuft, sem).wait()
        pltpu.sync_copy(ubuft, outu_h.at[pl.ds(b, ROW_TAIL), :])
        pltpu.sync_copy(ii_h.at[pl.ds(b, ROW_TAIL)], ivt)
        pltpu.async_copy(iemb_h.at[ivt], ubuft, sem).wait()
        pltpu.sync_copy(ubuft, outi_h.at[pl.ds(b, ROW_TAIL), :])


def _sc_embed(iu, ii, uemb, iemb):
    k = pl.kernel(
        _embed_body,
        out_type=(jax.ShapeDtypeStruct((N, EMB), jnp.float32),
                  jax.ShapeDtypeStruct((N, EMB), jnp.float32)),
        mesh=_mesh,
        scratch_types=[
            pltpu.VMEM((CH,), jnp.int32), pltpu.VMEM((CH, EMB), jnp.float32),
            pltpu.VMEM((ROW_TAIL,), jnp.int32), pltpu.VMEM((ROW_TAIL, EMB), jnp.float32),
            pltpu.SemaphoreType.DMA,
        ],
    )
    return k(iu, ii, uemb, iemb)


def _segsum_body(x_h, row_h, col_h, zer_h, out_h,
                 rv, cv, rvt, cvt, xbuf, xbuft, acc, sem):
    cid = lax.axis_index("c")
    sid = lax.axis_index("s")
    w = _wid()
    pltpu.sync_copy(zer_h, acc.at[pl.ds(sid * RPS, RPS), :])

    @pl.when(sid == NSUB - 1)
    def _z_tail():
        pltpu.sync_copy(zer_h.at[pl.ds(0, RTAIL), :], acc.at[pl.ds(NSUB * RPS, RTAIL), :])
    plsc.subcore_barrier()
    base_w = w * EPW

    def chunk(k, _):
        b = base_w + k * CH
        pltpu.sync_copy(row_h.at[pl.ds(b, CH)], rv)
        pltpu.sync_copy(col_h.at[pl.ds(b, CH)], cv)
        pltpu.async_copy(x_h.at[rv], xbuf, sem).wait()
        pltpu.sync_copy(xbuf, acc.at[cv], add=True)
        return _

    lax.fori_loop(0, FULL_CHUNKS, chunk, None)
    b = base_w + FULL_CHUNKS * CH
    pltpu.sync_copy(row_h.at[pl.ds(b, TAIL)], rvt)
    pltpu.sync_copy(col_h.at[pl.ds(b, TAIL)], cvt)
    pltpu.async_copy(x_h.at[rvt], xbuft, sem).wait()
    pltpu.sync_copy(xbuft, acc.at[cvt], add=True)
    plsc.subcore_barrier()
    r0 = sid * RPS
    pltpu.sync_copy(acc.at[pl.ds(r0, RPS), :], out_h.at[cid, pl.ds(r0, RPS), :])

    @pl.when(sid == NSUB - 1)
    def _o_tail():
        t0 = NSUB * RPS
        pltpu.sync_copy(acc.at[pl.ds(t0, RTAIL), :], out_h.at[cid, pl.ds(t0, RTAIL), :])


def _sc_segsum(x, row, col):
    zer = jnp.zeros((RPS, D), jnp.float32)
    k = pl.kernel(
        _segsum_body,
        out_type=jax.ShapeDtypeStruct((NCORE, N, D), jnp.float32),
        mesh=_mesh,
        scratch_types=[
            pltpu.VMEM((CH,), jnp.int32), pltpu.VMEM((CH,), jnp.int32),
            pltpu.VMEM((TAIL,), jnp.int32), pltpu.VMEM((TAIL,), jnp.int32),
            pltpu.VMEM((CH, D), jnp.float32), pltpu.VMEM((TAIL, D), jnp.float32),
            pltpu.VMEM_SHARED((N, D), jnp.float32),
            pltpu.SemaphoreType.DMA,
        ],
    )
    return k(x, row, col, zer)


def _gat_body(xw_h, als_h, ald_h, sv_h, row_h, col_h, zer_h, zer1_h, out_h, outd_h,
              rv, cv, rvt, cvt, xbuf, xbuft, obuf, obuft,
              asb, adb, asbt, adbt, sv_v, exb, bnc, acc, den, sem):
    cid = lax.axis_index("c")
    sid = lax.axis_index("s")
    w = _wid()
    pltpu.sync_copy(sv_h, sv_v)
    pltpu.sync_copy(zer_h, acc.at[pl.ds(sid * RPS, RPS), :])
    pltpu.sync_copy(zer1_h, bnc)
    pltpu.sync_copy(bnc, den.at[pl.ds(sid * RPS1, RPS1)])

    @pl.when(sid == NSUB - 1)
    def _z_tail():
        pltpu.sync_copy(zer_h.at[pl.ds(0, RTAIL), :], acc.at[pl.ds(NSUB * RPS, RTAIL), :])
    plsc.subcore_barrier()
    base_w = w * EPW
    svec = sv_v[...]

    def weight_chunk(n, asb_, adb_, xbuf_, obuf_):
        for j in range(n // 16):
            z = asb_[pl.ds(16 * j, 16)] + adb_[pl.ds(16 * j, 16)]
            lr = jnp.where(z > 0, z, 0.2 * z)
            exb[pl.ds(16 * j, 16)] = jnp.exp(lr - svec)

        def wrow(b, _):
            wv = exb[pl.ds(b, 16)][0]
            for j in range(D // 16):
                obuf_[b, pl.ds(16 * j, 16)] = xbuf_[b, pl.ds(16 * j, 16)] * wv
            return _

        lax.fori_loop(0, n, wrow, None)

    def chunk(k, _):
        b = base_w + k * CH
        pltpu.sync_copy(row_h.at[pl.ds(b, CH)], rv)
        pltpu.sync_copy(col_h.at[pl.ds(b, CH)], cv)
        pltpu.async_copy(als_h.at[rv], asb, sem).wait()
        pltpu.async_copy(ald_h.at[cv], adb, sem).wait()
        pltpu.async_copy(xw_h.at[rv], xbuf, sem).wait()
        weight_chunk(CH, asb, adb, xbuf, obuf)
        pltpu.sync_copy(obuf, acc.at[cv], add=True)
        pltpu.sync_copy(exb.at[pl.ds(0, CH)], den.at[cv], add=True)
        return _

    lax.fori_loop(0, FULL_CHUNKS, chunk, None)
    b = base_w + FULL_CHUNKS * CH
    pltpu.sync_copy(row_h.at[pl.ds(b, TAIL)], rvt)
    pltpu.sync_copy(col_h.at[pl.ds(b, TAIL)], cvt)
    pltpu.async_copy(als_h.at[rvt], asbt, sem).wait()
    pltpu.async_copy(ald_h.at[cvt], adbt, sem).wait()
    pltpu.async_copy(xw_h.at[rvt], xbuft, sem).wait()
    weight_chunk(TAIL, asbt, adbt, xbuft, obuft)
    pltpu.sync_copy(obuft, acc.at[cvt], add=True)
    pltpu.sync_copy(exb.at[pl.ds(0, TAIL)], den.at[cvt], add=True)
    plsc.subcore_barrier()
    r0 = sid * RPS
    pltpu.sync_copy(acc.at[pl.ds(r0, RPS), :], out_h.at[cid, pl.ds(r0, RPS), :])
    r1 = sid * RPS1
    pltpu.sync_copy(den.at[pl.ds(r1, RPS1)], bnc)
    pltpu.sync_copy(bnc, outd_h.at[cid, 0, pl.ds(r1, RPS1)])

    @pl.when(sid == NSUB - 1)
    def _o_tail():
        t0 = NSUB * RPS
        pltpu.sync_copy(acc.at[pl.ds(t0, RTAIL), :], out_h.at[cid, pl.ds(t0, RTAIL), :])


def _sc_gat(xw, als, ald, sv, row, col):
    zer = jnp.zeros((RPS, D), jnp.float32)
    zer1 = jnp.zeros((RPS1,), jnp.float32)
    k = pl.kernel(
        _gat_body,
        out_type=(jax.ShapeDtypeStruct((NCORE, N, D), jnp.float32),
                  jax.ShapeDtypeStruct((NCORE, 1, NP), jnp.float32)),
        mesh=_mesh,
        scratch_types=[
            pltpu.VMEM((CH,), jnp.int32), pltpu.VMEM((CH,), jnp.int32),
            pltpu.VMEM((TAIL,), jnp.int32), pltpu.VMEM((TAIL,), jnp.int32),
            pltpu.VMEM((CH, D), jnp.float32), pltpu.VMEM((TAIL, D), jnp.float32),
            pltpu.VMEM((CH, D), jnp.float32), pltpu.VMEM((TAIL, D), jnp.float32),
            pltpu.VMEM((CH,), jnp.float32), pltpu.VMEM((CH,), jnp.float32),
            pltpu.VMEM((TAIL,), jnp.float32), pltpu.VMEM((TAIL,), jnp.float32),
            pltpu.VMEM((16,), jnp.float32), pltpu.VMEM((CH + 16,), jnp.float32),
            pltpu.VMEM((RPS1,), jnp.float32),
            pltpu.VMEM_SHARED((N, D), jnp.float32),
            pltpu.VMEM_SHARED((NP,), jnp.float32),
            pltpu.SemaphoreType.DMA,
        ],
    )
    return k(xw, als, ald, sv, row, col, zer, zer1)


_TC_PARAMS = pltpu.CompilerParams(vmem_limit_bytes=100 * 1024 * 1024)


def _tc(body, out_shape):
    return pl.pallas_call(body, out_shape=out_shape, compiler_params=_TC_PARAMS)


def _cnt(ref):
    return ref[0, 0, :N] + ref[1, 0, :N]


def _gcn_pre_body(u_r, i_r, wg_r, cnt_r, xw_r, xs_r):
    xw = jnp.dot(u_r[...], wg_r[:EMB, :], preferred_element_type=jnp.float32)
    xw = xw + jnp.dot(i_r[...], wg_r[EMB:, :], preferred_element_type=jnp.float32)
    dis = lax.rsqrt(_cnt(cnt_r) + 1.0)
    xw_r[...] = xw
    xs_r[...] = xw * dis[:, None]


def _gcn_post_body(agg_r, xw_r, cnt_r, gb_r, bng_r, bnb_r, h_r):
    agg = agg_r[0] + agg_r[1]
    xw = xw_r[...]
    dis = lax.rsqrt(_cnt(cnt_r) + 1.0)
    pre = dis[:, None] * (agg + dis[:, None] * xw) + gb_r[...]
    mean = jnp.mean(pre, axis=0)
    var = jnp.mean((pre - mean) ** 2, axis=0)
    y = (pre - mean) * lax.rsqrt(var + 1e-5) * bng_r[...] + bnb_r[...]
    h_r[...] = jnp.maximum(y, 0.0)


def _sage_body(s_r, h_r, cntc_r, wl_r, bl_r, wr_r, cntr_r, h2_r, xs2_r):
    s = s_r[0] + s_r[1]
    h = h_r[...]
    mean = s / jnp.clip(_cnt(cntc_r), 1.0, None)[:, None]
    h2 = jnp.dot(mean, wl_r[...], preferred_element_type=jnp.float32) + bl_r[...]
    h2 = h2 + jnp.dot(h, wr_r[...], preferred_element_type=jnp.float32)
    h2 = jnp.maximum(h2, 0.0)
    cr = _cnt(cntr_r)
    disr = jnp.where(cr > 0, lax.rsqrt(jnp.maximum(cr, 1e-30)), 0.0)
    h2_r[...] = h2
    xs2_r[...] = h2 * disr[:, None]


def _cheb_gat_prep_body(t_r, h_r, cntr_r, w0_r, w1_r, cb_r, gw_r, gas_r, gad_r,
                        xw_r, als_r, ald_r, sv_r):
    cr = _cnt(cntr_r)
    disr = jnp.where(cr > 0, lax.rsqrt(jnp.maximum(cr, 1e-30)), 0.0)
    tx1 = -disr[:, None] * (t_r[0] + t_r[1])
    h = h_r[...]
    h3 = jnp.dot(h, w0_r[...], preferred_element_type=jnp.float32)
    h3 = h3 + jnp.dot(tx1, w1_r[...], preferred_element_type=jnp.float32) + cb_r[...]
    h3 = jnp.maximum(h3, 0.0)
    xw = jnp.dot(h3, gw_r[...], preferred_element_type=jnp.float32)
    als = jnp.dot(xw, gas_r[...][:, None], preferred_element_type=jnp.float32)[:, 0]
    ald = jnp.dot(xw, gad_r[...][:, None], preferred_element_type=jnp.float32)[:, 0]
    s = jnp.maximum(jnp.max(als) + jnp.max(ald), 0.0)
    xw_r[...] = xw
    als_r[...] = als
    ald_r[...] = ald
    sv_r[...] = jnp.broadcast_to(s, (16,))


def _gat_finish(acc_r, den_r, xw_r, als_r, ald_r, gb_r):
    feat = acc_r[0] + acc_r[1]
    den = den_r[0, 0, :N] + den_r[1, 0, :N]
    als = als_r[...]
    ald = ald_r[...]
    s = jnp.maximum(jnp.max(als) + jnp.max(ald), 0.0)
    zs = als + ald
    exs = jnp.exp(jnp.where(zs > 0, zs, 0.2 * zs) - s)
    xw = xw_r[...]
    out = (feat + exs[:, None] * xw) / (den + exs)[:, None] + gb_r[...]
    return jnp.where(out > 0, out, jnp.exp(jnp.minimum(out, 0.0)) - 1.0)


def _gat1_post_body(acc_r, den_r, xw_r, als_r, ald_r, gb_r, gw2_r, gas2_r, gad2_r,
                    xw2_r, als2_r, ald2_r, sv2_r):
    h4 = _gat_finish(acc_r, den_r, xw_r, als_r, ald_r, gb_r)
    xw2 = jnp.dot(h4, gw2_r[...], preferred_element_type=jnp.float32)
    als2 = jnp.dot(xw2, gas2_r[...][:, None], preferred_element_type=jnp.float32)[:, 0]
    ald2 = jnp.dot(xw2, gad2_r[...][:, None], preferred_element_type=jnp.float32)[:, 0]
    s2 = jnp.maximum(jnp.max(als2) + jnp.max(ald2), 0.0)
    xw2_r[...] = xw2
    als2_r[...] = als2
    ald2_r[...] = ald2
    sv2_r[...] = jnp.broadcast_to(s2, (16,))


def _gat2_post_body(acc_r, den_r, xw_r, als_r, ald_r, gb_r, h5_r):
    h5_r[...] = _gat_finish(acc_r, den_r, xw_r, als_r, ald_r, gb_r)


BC = 256


def _final_body(h_r, w_r, b_r, o_r):
    o_r[...] = jnp.dot(h_r[...], w_r[...], preferred_element_type=jnp.float32) + b_r[...]


def _final_matmul(h5, pred_W, pred_b):
    nb = (NC_OUT + BC - 1) // BC
    return pl.pallas_call(
        _final_body,
        grid=(nb,),
        in_specs=[
            pl.BlockSpec((N, D), lambda j: (0, 0)),
            pl.BlockSpec((D, BC), lambda j: (0, j)),
            pl.BlockSpec((BC,), lambda j: (j,)),
        ],
        out_specs=pl.BlockSpec((N, BC), lambda j: (0, j)),
        out_shape=jax.ShapeDtypeStruct((N, NC_OUT), jnp.float32),
        compiler_params=_TC_PARAMS,
    )(h5, pred_W, pred_b)


def kernel(x, edge_index, user_emb, item_emb, gcn_W, gcn_b, bn_g, bn_b,
           sage_Wl, sage_bl, sage_Wr, cheb_W0, cheb_W1, cheb_b,
           gat1_W, gat1_as, gat1_ad, gat1_b, gat2_W, gat2_as, gat2_ad, gat2_b,
           pred_W, pred_b):
    row = edge_index[0]
    col = edge_index[1]

    cntc, cntr = _sc_degrees(row, col)
    u, i = _sc_embed(x[:, 0], x[:, 1], user_emb, item_emb)

    xw, xs = _tc(_gcn_pre_body,
                 (jax.ShapeDtypeStruct((N, D), jnp.float32),
                  jax.ShapeDtypeStruct((N, D), jnp.float32)))(u, i, gcn_W, cntc)
    agg = _sc_segsum(xs, row, col)
    h1 = _tc(_gcn_post_body, jax.ShapeDtypeStruct((N, D), jnp.float32))(
        agg, xw, cntc, gcn_b, bn_g, bn_b)

    s = _sc_segsum(h1, row, col)
    h2, xs2 = _tc(_sage_body,
                  (jax.ShapeDtypeStruct((N, D), jnp.float32),
                   jax.ShapeDtypeStruct((N, D), jnp.float32)))(
        s, h1, cntc, sage_Wl, sage_bl, sage_Wr, cntr)

    t = _sc_segsum(xs2, row, col)
    xw1, als1, ald1, sv1 = _tc(
        _cheb_gat_prep_body,
        (jax.ShapeDtypeStruct((N, D), jnp.float32),
         jax.ShapeDtypeStruct((N,), jnp.float32),
         jax.ShapeDtypeStruct((N,), jnp.float32),
         jax.ShapeDtypeStruct((16,), jnp.float32)))(
        t, h2, cntr, cheb_W0, cheb_W1, cheb_b, gat1_W, gat1_as, gat1_ad)

    acc1, den1 = _sc_gat(xw1, als1, ald1, sv1, row, col)
    xw2, als2, ald2, sv2 = _tc(
        _gat1_post_body,
        (jax.ShapeDtypeStruct((N, D), jnp.float32),
         jax.ShapeDtypeStruct((N,), jnp.float32),
         jax.ShapeDtypeStruct((N,), jnp.float32),
         jax.ShapeDtypeStruct((16,), jnp.float32)))(
        acc1, den1, xw1, als1, ald1, gat1_b, gat2_W, gat2_as, gat2_ad)

    acc2, den2 = _sc_gat(xw2, als2, ald2, sv2, row, col)
    h5 = _tc(_gat2_post_body, jax.ShapeDtypeStruct((N, D), jnp.float32))(
        acc2, den2, xw2, als2, ald2, gat2_b)

    return _final_matmul(h5, pred_W, pred_b)
